# baseline (speedup 1.0000x reference)
import functools

import jax
import jax.numpy as jnp
from jax import lax
from jax.experimental import pallas as pl
from jax.experimental.pallas import tpu as pltpu
from jax.experimental.pallas import tpu_sc as plsc

_BATCH = 16384
_FEAT = 32
_LANES = 16

_NC = 2
_NS = 16
_NW = _NC * _NS
_BPW = _BATCH // _NW
_CHUNK = 128
_NCHUNK = _BPW // _CHUNK


@functools.cache
def _build():
    mesh = plsc.VectorSubcoreMesh(core_axis_name="c", subcore_axis_name="s")

    @functools.partial(
        pl.kernel,
        mesh=mesh,
        out_type=jax.ShapeDtypeStruct((_NW, _LANES), jnp.float32),
        scratch_types=[
            pltpu.VMEM((_NCHUNK, _CHUNK), jnp.int32),
            pltpu.VMEM((_BPW, _FEAT), jnp.float32),
            pltpu.VMEM((_BPW, _FEAT), jnp.float32),
            pltpu.VMEM((_LANES,), jnp.float32),
            pltpu.SemaphoreType.DMA,
            pltpu.SemaphoreType.DMA,
        ],
        compiler_params=pltpu.CompilerParams(use_tc_tiling_on_sc=False),
    )
    def center_loss_partials(features_hbm, labels_hbm, centers_hbm, out_hbm,
                             idx_v, ctr_v, feat_v, acc_v, gsem, fsem):
        wid = lax.axis_index("s") * _NC + lax.axis_index("c")
        base = wid * _BPW

        pltpu.sync_copy(labels_hbm.at[wid], idx_v)
        fcopy = pltpu.async_copy(
            features_hbm.at[pl.ds(base, _BPW)], feat_v, fsem)
        gcopies = [
            pltpu.async_copy(
                centers_hbm.at[idx_v.at[j]],
                ctr_v.at[pl.ds(j * _CHUNK, _CHUNK)],
                gsem)
            for j in range(_NCHUNK)
        ]
        fcopy.wait()
        for c in gcopies:
            c.wait()

        zeros = jnp.zeros((_LANES,), jnp.float32)

        def step(i, carry):
            a0, a1 = carry
            f0 = feat_v[i, pl.ds(0, _LANES)]
            f1 = feat_v[i, pl.ds(_LANES, _LANES)]
            c0 = ctr_v[i, pl.ds(0, _LANES)]
            c1 = ctr_v[i, pl.ds(_LANES, _LANES)]
            d0 = f0 - c0
            d1 = f1 - c1
            return a0 + d0 * d0, a1 + d1 * d1

        a0, a1 = lax.fori_loop(0, _BPW, step, (zeros, zeros), unroll=4)
        acc_v[...] = a0 + a1
        pltpu.sync_copy(acc_v, out_hbm.at[wid])

    return center_loss_partials


def kernel(features, labels, centers):
    labels = labels.astype(jnp.int32).reshape(_NW, _NCHUNK, _CHUNK)
    partials = _build()(features, labels, centers)
    return jnp.sum(partials) * (0.5 / _BATCH)
